# baseline (speedup 1.0000x reference)
import functools

import jax
import jax.numpy as jnp
from jax import lax
from jax.experimental import pallas as pl
from jax.experimental.pallas import tpu as pltpu
from jax.experimental.pallas import tpu_sc as plsc

NC = 2
NS = 16
L = 16
NW = NC * NS

CIN = 512
C = 128
NB = 8


@functools.partial(jax.jit, static_argnames=("n", "f_rows", "u_rows", "d"))
def _sc_lookup(idx_flat, weight_frozen, weight_unfrozen, *, n, f_rows, u_rows, d):
    n_w = n // NW
    mesh = plsc.VectorSubcoreMesh(core_axis_name="c", subcore_axis_name="s")

    @functools.partial(
        pl.kernel,
        out_type=jax.ShapeDtypeStruct((n, d), jnp.float32),
        mesh=mesh,
        scratch_types=[
            pltpu.VMEM((2, CIN), jnp.int32),
            pltpu.VMEM((n_w,), jnp.int32),
            pltpu.VMEM((n_w,), jnp.int32),
            pltpu.VMEM((NB, C), jnp.int32),
            pltpu.VMEM((NB, C), jnp.int32),
            pltpu.VMEM((NB, C, d), jnp.float32),
            pltpu.SemaphoreType.DMA,
            pltpu.SemaphoreType.DMA,
        ] + [pltpu.SemaphoreType.DMA] * (2 * NB),
        compiler_params=pltpu.CompilerParams(
            needs_layout_passes=False, use_tc_tiling_on_sc=False),
    )
    def k(idx_hbm, frozen_hbm, unfrozen_hbm, out_hbm,
          inbuf, idx_buf, pos_buf, sidx, spos, rows,
          in_sem0, in_sem1, *gs_sems):
        in_sems = (in_sem0, in_sem1)
        g_sems = gs_sems[:NB]
        s_sems = gs_sems[NB:]
        wid = lax.axis_index("s") * NC + lax.axis_index("c")
        base = wid * n_w
        iot = lax.iota(jnp.int32, L)
        zeros16 = jnp.zeros((L,), jnp.int32)

        def p1_copy(kk, b):
            return pltpu.make_async_copy(
                idx_hbm.at[pl.ds(base + kk * CIN, CIN)], inbuf.at[b],
                in_sems[b])

        def p1_process(b, kk, off_f, off_u):
            for t in range(CIN // L):
                v = inbuf[b, pl.ds(t * L, L)]
                gpos = base + kk * CIN + t * L + iot
                m = v < f_rows
                mi = m.astype(jnp.int32)
                s = plsc.cumsum(mi)
                nf_t = jnp.sum(mi)
                dstf = off_f + (s - mi)
                dstu = (n_w - 1) - (off_u + (iot + mi - s))
                plsc.store_scatter(idx_buf, [dstf], v, mask=m)
                plsc.store_scatter(pos_buf, [dstf], gpos, mask=m)
                plsc.store_scatter(idx_buf, [dstu], v - f_rows, mask=~m)
                plsc.store_scatter(pos_buf, [dstu], gpos, mask=~m)
                off_f = off_f + nf_t
                off_u = off_u + (L - nf_t)
            return off_f, off_u

        nsteps = n_w // (2 * CIN)
        p1_copy(0, 0).start()

        def p1_step(ss, carry):
            off_f, off_u = carry
            k0 = 2 * ss
            p1_copy(k0 + 1, 1).start()
            p1_copy(k0, 0).wait()
            off_f, off_u = p1_process(0, k0, off_f, off_u)

            @pl.when(ss < nsteps - 1)
            def _():
                p1_copy(k0 + 2, 0).start()

            p1_copy(k0 + 1, 1).wait()
            off_f, off_u = p1_process(1, k0 + 1, off_f, off_u)
            return off_f, off_u

        n_f, _ = lax.fori_loop(0, nsteps, p1_step, (0, 0))

        def run_list(table_hbm, nch, frozen_side):
            first = 0 if frozen_side else n_w - 1
            iv0 = plsc.load_gather(idx_buf, [zeros16 + first])
            pv0 = plsc.load_gather(pos_buf, [zeros16 + first])

            def stage(j, b):
                for t in range(C // L):
                    if frozen_side:
                        off = j * C + t * L
                        valid = (off + iot) < n_f
                    else:
                        off = (n_w - (j + 1) * C) + t * L
                        valid = (off + iot) >= n_f
                    iv = idx_buf[pl.ds(off, L)]
                    pv = pos_buf[pl.ds(off, L)]
                    sidx[b, pl.ds(t * L, L)] = jnp.where(valid, iv, iv0)
                    spos[b, pl.ds(t * L, L)] = jnp.where(valid, pv, pv0)

            def scat(b):
                return pltpu.make_async_copy(
                    rows.at[b], out_hbm.at[spos.at[b]], s_sems[b])

            def step(s, _):
                for b in range(NB):
                    j = NB * s + b

                    @pl.when(j < nch)
                    def _(j=j, b=b):
                        @pl.when(s > 0)
                        def _():
                            scat(b).wait()

                        stage(j, b)
                        pltpu.make_async_copy(
                            table_hbm.at[sidx.at[b]], rows.at[b], g_sems[b]
                        ).start()
                for b in range(NB):

                    @pl.when(NB * s + b < nch)
                    def _(b=b):
                        pltpu.make_async_copy(
                            table_hbm.at[sidx.at[b]], rows.at[b], g_sems[b]
                        ).wait()
                        scat(b).start()
                return 0

            lax.fori_loop(0, (nch + NB - 1) // NB, step, 0)
            for b in range(NB):

                @pl.when(b < nch)
                def _(b=b):
                    scat(b).wait()

        run_list(frozen_hbm, (n_f + C - 1) // C, True)
        run_list(unfrozen_hbm, (n_w - n_f + C - 1) // C, False)

    return k(idx_flat, weight_frozen, weight_unfrozen)


def kernel(idx, weight_frozen, weight_unfrozen):
    f_rows, d = weight_frozen.shape
    u_rows = weight_unfrozen.shape[0]
    n = idx.shape[0] * idx.shape[1]
    out = _sc_lookup(idx.reshape(-1), weight_frozen, weight_unfrozen,
                     n=n, f_rows=f_rows, u_rows=u_rows, d=d)
    return out.reshape(idx.shape + (d,))
